# baseline (speedup 1.0000x reference)
import functools

import numpy as np
import jax
import jax.numpy as jnp
from jax import lax
from jax.experimental import pallas as pl
from jax.experimental.pallas import tpu as pltpu
from jax.experimental.pallas import tpu_sc as plsc

_B, _D = 16384, 256
_T, _P = 26, 128
_K = _T * _P
_COMMIT = 0.25
_TEMP = 0.07
_TILE = 256
_G = _B // _TILE

_SAMP = int(0.1 * _P)
_N = _T * _SAMP
_NPAD = 320

_PERM12 = np.array([121, 35, 45, 99, 31, 112, 85, 63, 117, 114, 82, 65],
                   dtype=np.int32)


def _samp_const():
    samp = (np.arange(_T, dtype=np.int32)[:, None] * _P
            + _PERM12[None, :]).reshape(1, _N)
    return np.concatenate(
        [samp, np.full((1, _NPAD - _N), -1)], axis=1).astype(np.int32)


def _dist_body(x_ref, q_ref, emb2_ref, idx_ref, dsum_ref, e2_ref):
    @pl.when(pl.program_id(0) == 0)
    def _():
        embT = emb2_ref[...] * 0.5
        e2_ref[...] = jnp.sum(embT * embT, axis=1, keepdims=True)
        dsum_ref[...] = jnp.zeros_like(dsum_ref)

    x = x_ref[...]
    xs = jnp.sum(x * x, axis=1, keepdims=True)
    xn = x / jnp.maximum(jnp.sqrt(xs), 1e-12)
    s2T = lax.dot_general(emb2_ref[...], xn,
                          (((1,), (1,)), ((), ())))
    xn2T = jnp.sum(xn * xn, axis=1, keepdims=True).reshape(1, _TILE)
    q = q_ref[0]
    slocT = jnp.zeros((_P, _TILE), jnp.float32)
    e2locT = jnp.zeros((_P, _TILE), jnp.float32)
    for t in range(_T):
        oh = (q == t).astype(jnp.float32)
        sl = slice(t * _P, (t + 1) * _P)
        slocT = slocT + oh * s2T[sl, :]
        e2locT = e2locT + oh * e2_ref[sl, :]
    dlocT = (xn2T + e2locT) - slocT
    dminT = jnp.min(dlocT, axis=0, keepdims=True)
    liT = lax.broadcasted_iota(jnp.int32, (_P, _TILE), 0)
    lidxT = jnp.min(jnp.where(dlocT == dminT, liT, _P), axis=0)
    idx_ref[0, 0, :] = q[0, :] * _P + lidxT
    dsum_ref[...] += jnp.sum(dminT, axis=1, keepdims=True)


def _make_dist(interpret=False):
    return pl.pallas_call(
        _dist_body,
        grid=(_G,),
        in_specs=[
            pl.BlockSpec((_TILE, _D), lambda i: (i, 0)),
            pl.BlockSpec((1, 1, _TILE), lambda i: (i, 0, 0)),
            pl.BlockSpec((_K, _D), lambda i: (0, 0)),
        ],
        out_specs=[
            pl.BlockSpec((1, 1, _TILE), lambda i: (i, 0, 0)),
            pl.BlockSpec((1, 1), lambda i: (0, 0)),
        ],
        out_shape=[
            jax.ShapeDtypeStruct((_G, 1, _TILE), jnp.int32),
            jax.ShapeDtypeStruct((1, 1), jnp.float32),
        ],
        scratch_shapes=[
            pltpu.VMEM((_K, 1), jnp.float32),
        ],
        interpret=interpret,
    )


def _unif_body(emb_ref, samp_ref, out_ref):
    samp = samp_ref[0, :]
    kiota = lax.broadcasted_iota(jnp.int32, (_NPAD, _K), 1)
    sel = (samp[:, None] == kiota).astype(jnp.float32)
    labels = samp // _P
    lab = ((labels[:, None] == labels[None, :])
           & (samp[:, None] >= 0) & (samp[None, :] >= 0)).astype(jnp.float32)
    se = lax.dot_general(sel, emb_ref[...], (((1,), (0,)), ((), ())),
                         precision=lax.Precision.HIGHEST)
    n = jnp.sqrt(jnp.sum(se * se, axis=1, keepdims=True))
    sen = se / jnp.maximum(n, 1e-12)
    sim = lax.dot_general(sen, sen, (((1,), (1,)), ((), ())))
    ri = lax.broadcasted_iota(jnp.int32, (_NPAD, _NPAD), 0)
    ci = lax.broadcasted_iota(jnp.int32, (_NPAD, _NPAD), 1)
    off = (ri != ci) & (ci < _N)
    e = jnp.exp(jnp.where(off, sim, -1e30) / _TEMP)
    sum_exp = jnp.sum(e, axis=1, keepdims=True)
    pos = jnp.sum(e * lab, axis=1, keepdims=True)
    rowv = lax.broadcasted_iota(jnp.int32, (_NPAD, 1), 0) < _N
    r = jnp.where(rowv, pos / sum_exp, 1.0)
    out_ref[...] = -jnp.sum(jnp.log(r), axis=0, keepdims=True) / _N


def _make_unif(interpret=False):
    return pl.pallas_call(
        _unif_body,
        out_shape=jax.ShapeDtypeStruct((1, 1), jnp.float32),
        interpret=interpret,
    )


_NC, _NS = 2, 16
_NW = _NC * _NS
_BPW = _B // _NW
_CH = 128
_NCHUNK = _BPW // _CH


@functools.cache
def _make_sc_gather():
    @functools.partial(
        pl.kernel,
        mesh=plsc.VectorSubcoreMesh(core_axis_name="c", subcore_axis_name="s"),
        out_type=jax.ShapeDtypeStruct((_B, _D), jnp.float32),
        scratch_types=[
            pltpu.VMEM((_CH,), jnp.int32),
            pltpu.VMEM((_CH, _D), jnp.float32),
            pltpu.SemaphoreType.DMA,
        ],
    )
    def _sc_gather(emb_hbm, idx_hbm, out_hbm, idx_v, rows_v, sem):
        wid = lax.axis_index("s") * _NC + lax.axis_index("c")
        base = wid * _BPW
        for c in range(_NCHUNK):
            off = base + c * _CH
            pltpu.sync_copy(idx_hbm.at[pl.ds(off, _CH)], idx_v)
            pltpu.async_copy(emb_hbm.at[idx_v], rows_v, sem).wait()
            pltpu.sync_copy(rows_v, out_hbm.at[pl.ds(off, _CH)])

    return _sc_gather


_dist = _make_dist()
_unif = _make_unif()


def kernel(x, Q, embeddings):
    idx3, dsum = _dist(x, Q.reshape(_G, 1, _TILE), embeddings * 2.0)
    idx = idx3.reshape(_B)
    quantized = _make_sc_gather()(embeddings, idx)
    loss = (1.0 + _COMMIT) * dsum[0, 0] / (_B * _D)
    ul = _unif(embeddings, _samp_const())[0, 0]
    return quantized, loss, ul, idx


# speedup vs baseline: 6.5375x; 6.5375x over previous
import functools

import numpy as np
import jax
import jax.numpy as jnp
from jax import lax
from jax.experimental import pallas as pl
from jax.experimental.pallas import tpu as pltpu
from jax.experimental.pallas import tpu_sc as plsc

_B, _D = 16384, 256
_T, _P = 26, 128
_K = _T * _P
_COMMIT = 0.25
_TEMP = 0.07
_TILE = 256
_G = _B // _TILE

_SAMP = int(0.1 * _P)
_N = _T * _SAMP
_NPAD = 320

_PERM12 = np.array([121, 35, 45, 99, 31, 112, 85, 63, 117, 114, 82, 65],
                   dtype=np.int32)


def _samp_const():
    samp = (np.arange(_T, dtype=np.int32)[:, None] * _P
            + _PERM12[None, :]).reshape(1, _N)
    return np.concatenate(
        [samp, np.full((1, _NPAD - _N), -1)], axis=1).astype(np.int32)


def _dist_body(x_ref, q_ref, embT2_ref, idx_ref, dsum_ref, e2_ref):
    @pl.when(pl.program_id(0) == 0)
    def _():
        embT = embT2_ref[...] * 0.5
        e2_ref[...] = jnp.sum(embT * embT, axis=0, keepdims=True)
        dsum_ref[...] = jnp.zeros_like(dsum_ref)

    x = x_ref[...]
    xs = jnp.sum(x * x, axis=1, keepdims=True)
    xn = x / jnp.maximum(jnp.sqrt(xs), 1e-12)
    s2 = jnp.dot(xn, embT2_ref[...])
    xn2 = jnp.sum(xn * xn, axis=1, keepdims=True)
    q = q_ref[0, 0, :]
    sloc = jnp.zeros((_TILE, _P), jnp.float32)
    e2loc = jnp.zeros((_TILE, _P), jnp.float32)
    for t in range(_T):
        oh = (q[:, None] == t).astype(jnp.float32)
        sl = slice(t * _P, (t + 1) * _P)
        sloc = sloc + oh * s2[:, sl]
        e2loc = e2loc + oh * e2_ref[:, sl]
    dloc = (xn2 + e2loc) - sloc
    dmin = jnp.min(dloc, axis=1, keepdims=True)
    li = lax.broadcasted_iota(jnp.int32, (_TILE, _P), 1)
    lidx = jnp.min(jnp.where(dloc == dmin, li, _P), axis=1)
    idx_ref[0, 0, :] = q * _P + lidx
    dsum_ref[...] += jnp.sum(dmin, axis=0, keepdims=True)


def _make_dist(interpret=False):
    return pl.pallas_call(
        _dist_body,
        grid=(_G,),
        in_specs=[
            pl.BlockSpec((_TILE, _D), lambda i: (i, 0)),
            pl.BlockSpec((1, 1, _TILE), lambda i: (i, 0, 0)),
            pl.BlockSpec((_D, _K), lambda i: (0, 0)),
        ],
        out_specs=[
            pl.BlockSpec((1, 1, _TILE), lambda i: (i, 0, 0)),
            pl.BlockSpec((1, 1), lambda i: (0, 0)),
        ],
        out_shape=[
            jax.ShapeDtypeStruct((_G, 1, _TILE), jnp.int32),
            jax.ShapeDtypeStruct((1, 1), jnp.float32),
        ],
        scratch_shapes=[
            pltpu.VMEM((1, _K), jnp.float32),
        ],
        interpret=interpret,
    )


def _unif_body(emb_ref, samp_ref, out_ref):
    samp = samp_ref[0, :]
    kiota = lax.broadcasted_iota(jnp.int32, (_NPAD, _K), 1)
    sel = (samp[:, None] == kiota).astype(jnp.float32)
    labels = samp // _P
    lab = ((labels[:, None] == labels[None, :])
           & (samp[:, None] >= 0) & (samp[None, :] >= 0)).astype(jnp.float32)
    se = lax.dot_general(sel, emb_ref[...], (((1,), (0,)), ((), ())),
                         precision=lax.Precision.HIGHEST)
    n = jnp.sqrt(jnp.sum(se * se, axis=1, keepdims=True))
    sen = se / jnp.maximum(n, 1e-12)
    sim = lax.dot_general(sen, sen, (((1,), (1,)), ((), ())))
    ri = lax.broadcasted_iota(jnp.int32, (_NPAD, _NPAD), 0)
    ci = lax.broadcasted_iota(jnp.int32, (_NPAD, _NPAD), 1)
    off = (ri != ci) & (ci < _N)
    e = jnp.exp(jnp.where(off, sim, -1e30) / _TEMP)
    sum_exp = jnp.sum(e, axis=1, keepdims=True)
    pos = jnp.sum(e * lab, axis=1, keepdims=True)
    rowv = lax.broadcasted_iota(jnp.int32, (_NPAD, 1), 0) < _N
    r = jnp.where(rowv, pos / sum_exp, 1.0)
    out_ref[...] = -jnp.sum(jnp.log(r), axis=0, keepdims=True) / _N


def _make_unif(interpret=False):
    return pl.pallas_call(
        _unif_body,
        out_shape=jax.ShapeDtypeStruct((1, 1), jnp.float32),
        interpret=interpret,
    )


_NC, _NS = 2, 16
_NW = _NC * _NS
_BPW = _B // _NW
_CH = 128
_NCHUNK = _BPW // _CH


@functools.cache
def _make_sc_gather():
    @functools.partial(
        pl.kernel,
        mesh=plsc.VectorSubcoreMesh(core_axis_name="c", subcore_axis_name="s"),
        out_type=jax.ShapeDtypeStruct((_B, _D), jnp.float32),
        scratch_types=[
            pltpu.VMEM((_CH,), jnp.int32),
            pltpu.VMEM((_CH,), jnp.int32),
            pltpu.VMEM((_CH, _D), jnp.float32),
            pltpu.VMEM((_CH, _D), jnp.float32),
            pltpu.SemaphoreType.DMA,
            pltpu.SemaphoreType.DMA,
        ],
    )
    def _sc_gather(emb_hbm, idx_hbm, out_hbm, i0, i1, r0, r1, s0, s1):
        wid = lax.axis_index("s") * _NC + lax.axis_index("c")
        base = wid * _BPW
        ib, rb, sb = (i0, i1), (r0, r1), (s0, s1)
        handles = {}
        pltpu.sync_copy(idx_hbm.at[pl.ds(base, _CH)], ib[0])
        handles[0] = pltpu.async_copy(emb_hbm.at[ib[0]], rb[0], sb[0])
        for c in range(_NCHUNK):
            r = c % 2
            if c + 1 < _NCHUNK:
                nr = (c + 1) % 2
                pltpu.sync_copy(
                    idx_hbm.at[pl.ds(base + (c + 1) * _CH, _CH)], ib[nr])
                handles[c + 1] = pltpu.async_copy(
                    emb_hbm.at[ib[nr]], rb[nr], sb[nr])
            handles[c].wait()
            pltpu.sync_copy(rb[r], out_hbm.at[pl.ds(base + c * _CH, _CH)])

    return _sc_gather


_dist = _make_dist()
_unif = _make_unif()


def kernel(x, Q, embeddings):
    idx3, dsum = _dist(x, Q.reshape(_G, 1, _TILE), embeddings.T * 2.0)
    idx = idx3.reshape(_B)
    quantized = _make_sc_gather()(embeddings, idx)
    loss = (1.0 + _COMMIT) * dsum[0, 0] / (_B * _D)
    ul = _unif(embeddings, _samp_const())[0, 0]
    return quantized, loss, ul, idx


# speedup vs baseline: 7.0929x; 1.0850x over previous
import functools

import numpy as np
import jax
import jax.numpy as jnp
from jax import lax
from jax.experimental import pallas as pl
from jax.experimental.pallas import tpu as pltpu
from jax.experimental.pallas import tpu_sc as plsc

_B, _D = 16384, 256
_T, _P = 26, 128
_K = _T * _P
_COMMIT = 0.25
_TEMP = 0.07
_TILE = 512
_G = _B // _TILE

_SAMP = int(0.1 * _P)
_N = _T * _SAMP
_NPAD = 320

_PERM12 = np.array([121, 35, 45, 99, 31, 112, 85, 63, 117, 114, 82, 65],
                   dtype=np.int32)


def _samp_const():
    samp = (np.arange(_T, dtype=np.int32)[:, None] * _P
            + _PERM12[None, :]).reshape(1, _N)
    return np.concatenate(
        [samp, np.full((1, _NPAD - _N), -1)], axis=1).astype(np.int32)


def _dist_body(x_ref, q_ref, embT2_ref, idx_ref, dsum_ref, e2_ref):
    @pl.when(pl.program_id(0) == 0)
    def _():
        embT = embT2_ref[...] * 0.5
        e2_ref[...] = jnp.sum(embT * embT, axis=0, keepdims=True)
        dsum_ref[...] = jnp.zeros_like(dsum_ref)

    x = x_ref[...]
    xs = jnp.sum(x * x, axis=1, keepdims=True)
    xn = x / jnp.maximum(jnp.sqrt(xs), 1e-12)
    s2 = jnp.dot(xn, embT2_ref[...])
    xn2 = jnp.sum(xn * xn, axis=1, keepdims=True)
    q = q_ref[0, 0, :]
    sloc = jnp.zeros((_TILE, _P), jnp.float32)
    e2loc = jnp.zeros((_TILE, _P), jnp.float32)
    for t in range(_T):
        oh = (q[:, None] == t).astype(jnp.float32)
        sl = slice(t * _P, (t + 1) * _P)
        sloc = sloc + oh * s2[:, sl]
        e2loc = e2loc + oh * e2_ref[:, sl]
    dloc = (xn2 + e2loc) - sloc
    dmin = jnp.min(dloc, axis=1, keepdims=True)
    li = lax.broadcasted_iota(jnp.int32, (_TILE, _P), 1)
    lidx = jnp.min(jnp.where(dloc == dmin, li, _P), axis=1)
    idx_ref[0, 0, :] = q * _P + lidx
    dsum_ref[...] += jnp.sum(dmin, axis=0, keepdims=True)


def _make_dist(interpret=False):
    return pl.pallas_call(
        _dist_body,
        grid=(_G,),
        in_specs=[
            pl.BlockSpec((_TILE, _D), lambda i: (i, 0)),
            pl.BlockSpec((1, 1, _TILE), lambda i: (i, 0, 0)),
            pl.BlockSpec((_D, _K), lambda i: (0, 0)),
        ],
        out_specs=[
            pl.BlockSpec((1, 1, _TILE), lambda i: (i, 0, 0)),
            pl.BlockSpec((1, 1), lambda i: (0, 0)),
        ],
        out_shape=[
            jax.ShapeDtypeStruct((_G, 1, _TILE), jnp.int32),
            jax.ShapeDtypeStruct((1, 1), jnp.float32),
        ],
        scratch_shapes=[
            pltpu.VMEM((1, _K), jnp.float32),
        ],
        interpret=interpret,
    )


def _unif_body(emb_ref, samp_ref, out_ref):
    samp = samp_ref[0, :]
    kiota = lax.broadcasted_iota(jnp.int32, (_NPAD, _K), 1)
    sel = (samp[:, None] == kiota).astype(jnp.float32)
    labels = samp // _P
    lab = ((labels[:, None] == labels[None, :])
           & (samp[:, None] >= 0) & (samp[None, :] >= 0)).astype(jnp.float32)
    se = lax.dot_general(sel, emb_ref[...], (((1,), (0,)), ((), ())),
                         precision=lax.Precision.HIGHEST)
    n = jnp.sqrt(jnp.sum(se * se, axis=1, keepdims=True))
    sen = se / jnp.maximum(n, 1e-12)
    sim = lax.dot_general(sen, sen, (((1,), (1,)), ((), ())))
    ri = lax.broadcasted_iota(jnp.int32, (_NPAD, _NPAD), 0)
    ci = lax.broadcasted_iota(jnp.int32, (_NPAD, _NPAD), 1)
    off = (ri != ci) & (ci < _N)
    e = jnp.exp(jnp.where(off, sim, -1e30) / _TEMP)
    sum_exp = jnp.sum(e, axis=1, keepdims=True)
    pos = jnp.sum(e * lab, axis=1, keepdims=True)
    rowv = lax.broadcasted_iota(jnp.int32, (_NPAD, 1), 0) < _N
    r = jnp.where(rowv, pos / sum_exp, 1.0)
    out_ref[...] = -jnp.sum(jnp.log(r), axis=0, keepdims=True) / _N


def _make_unif(interpret=False):
    return pl.pallas_call(
        _unif_body,
        out_shape=jax.ShapeDtypeStruct((1, 1), jnp.float32),
        interpret=interpret,
    )


_NC, _NS = 2, 16
_NW = _NC * _NS
_BPW = _B // _NW
_CH = 128
_NCHUNK = _BPW // _CH


@functools.cache
def _make_sc_gather():
    @functools.partial(
        pl.kernel,
        mesh=plsc.VectorSubcoreMesh(core_axis_name="c", subcore_axis_name="s"),
        out_type=jax.ShapeDtypeStruct((_B, _D), jnp.float32),
        scratch_types=[
            pltpu.VMEM((_CH,), jnp.int32),
            pltpu.VMEM((_CH,), jnp.int32),
            pltpu.VMEM((_CH, _D), jnp.float32),
            pltpu.VMEM((_CH, _D), jnp.float32),
            pltpu.SemaphoreType.DMA,
            pltpu.SemaphoreType.DMA,
        ],
    )
    def _sc_gather(emb_hbm, idx_hbm, out_hbm, i0, i1, r0, r1, s0, s1):
        wid = lax.axis_index("s") * _NC + lax.axis_index("c")
        base = wid * _BPW
        ib, rb, sb = (i0, i1), (r0, r1), (s0, s1)
        handles = {}
        pltpu.sync_copy(idx_hbm.at[pl.ds(base, _CH)], ib[0])
        handles[0] = pltpu.async_copy(emb_hbm.at[ib[0]], rb[0], sb[0])
        for c in range(_NCHUNK):
            r = c % 2
            if c + 1 < _NCHUNK:
                nr = (c + 1) % 2
                pltpu.sync_copy(
                    idx_hbm.at[pl.ds(base + (c + 1) * _CH, _CH)], ib[nr])
                handles[c + 1] = pltpu.async_copy(
                    emb_hbm.at[ib[nr]], rb[nr], sb[nr])
            handles[c].wait()
            pltpu.sync_copy(rb[r], out_hbm.at[pl.ds(base + c * _CH, _CH)])

    return _sc_gather


_dist = _make_dist()
_unif = _make_unif()


def kernel(x, Q, embeddings):
    idx3, dsum = _dist(x, Q.reshape(_G, 1, _TILE), embeddings.T * 2.0)
    idx = idx3.reshape(_B)
    quantized = _make_sc_gather()(embeddings, idx)
    loss = (1.0 + _COMMIT) * dsum[0, 0] / (_B * _D)
    ul = _unif(embeddings, _samp_const())[0, 0]
    return quantized, loss, ul, idx


# speedup vs baseline: 7.1746x; 1.0115x over previous
import functools

import numpy as np
import jax
import jax.numpy as jnp
from jax import lax
from jax.experimental import pallas as pl
from jax.experimental.pallas import tpu as pltpu
from jax.experimental.pallas import tpu_sc as plsc

_B, _D = 16384, 256
_T, _P = 26, 128
_K = _T * _P
_COMMIT = 0.25
_TEMP = 0.07
_TILE = 1024
_G = _B // _TILE

_SAMP = int(0.1 * _P)
_N = _T * _SAMP
_NPAD = 320

_PERM12 = np.array([121, 35, 45, 99, 31, 112, 85, 63, 117, 114, 82, 65],
                   dtype=np.int32)


def _samp_const():
    samp = (np.arange(_T, dtype=np.int32)[:, None] * _P
            + _PERM12[None, :]).reshape(1, _N)
    return np.concatenate(
        [samp, np.full((1, _NPAD - _N), -1)], axis=1).astype(np.int32)


def _dist_body(x_ref, q_ref, embT2_ref, idx_ref, dsum_ref, e2_ref):
    @pl.when(pl.program_id(0) == 0)
    def _():
        embT = embT2_ref[...] * 0.5
        e2_ref[...] = jnp.sum(embT * embT, axis=0, keepdims=True)
        dsum_ref[...] = jnp.zeros_like(dsum_ref)

    x = x_ref[...]
    xs = jnp.sum(x * x, axis=1, keepdims=True)
    xn = x / jnp.maximum(jnp.sqrt(xs), 1e-12)
    s2 = jnp.dot(xn, embT2_ref[...])
    xn2 = jnp.sum(xn * xn, axis=1, keepdims=True)
    q = q_ref[0, 0, :]
    sloc = jnp.zeros((_TILE, _P), jnp.float32)
    e2loc = jnp.zeros((_TILE, _P), jnp.float32)
    for t in range(_T):
        oh = (q[:, None] == t).astype(jnp.float32)
        sl = slice(t * _P, (t + 1) * _P)
        sloc = sloc + oh * s2[:, sl]
        e2loc = e2loc + oh * e2_ref[:, sl]
    dloc = (xn2 + e2loc) - sloc
    dmin = jnp.min(dloc, axis=1, keepdims=True)
    li = lax.broadcasted_iota(jnp.int32, (_TILE, _P), 1)
    lidx = jnp.min(jnp.where(dloc == dmin, li, _P), axis=1)
    idx_ref[0, 0, :] = q * _P + lidx
    dsum_ref[...] += jnp.sum(dmin, axis=0, keepdims=True)


def _make_dist(interpret=False):
    return pl.pallas_call(
        _dist_body,
        grid=(_G,),
        in_specs=[
            pl.BlockSpec((_TILE, _D), lambda i: (i, 0)),
            pl.BlockSpec((1, 1, _TILE), lambda i: (i, 0, 0)),
            pl.BlockSpec((_D, _K), lambda i: (0, 0)),
        ],
        out_specs=[
            pl.BlockSpec((1, 1, _TILE), lambda i: (i, 0, 0)),
            pl.BlockSpec((1, 1), lambda i: (0, 0)),
        ],
        out_shape=[
            jax.ShapeDtypeStruct((_G, 1, _TILE), jnp.int32),
            jax.ShapeDtypeStruct((1, 1), jnp.float32),
        ],
        scratch_shapes=[
            pltpu.VMEM((1, _K), jnp.float32),
        ],
        interpret=interpret,
    )


def _unif_body(emb_ref, samp_ref, out_ref):
    samp = samp_ref[0, :]
    kiota = lax.broadcasted_iota(jnp.int32, (_NPAD, _K), 1)
    sel = (samp[:, None] == kiota).astype(jnp.float32)
    labels = samp // _P
    lab = ((labels[:, None] == labels[None, :])
           & (samp[:, None] >= 0) & (samp[None, :] >= 0)).astype(jnp.float32)
    se = lax.dot_general(sel, emb_ref[...], (((1,), (0,)), ((), ())),
                         precision=lax.Precision.HIGHEST)
    n = jnp.sqrt(jnp.sum(se * se, axis=1, keepdims=True))
    sen = se / jnp.maximum(n, 1e-12)
    sim = lax.dot_general(sen, sen, (((1,), (1,)), ((), ())))
    ri = lax.broadcasted_iota(jnp.int32, (_NPAD, _NPAD), 0)
    ci = lax.broadcasted_iota(jnp.int32, (_NPAD, _NPAD), 1)
    off = (ri != ci) & (ci < _N)
    e = jnp.exp(jnp.where(off, sim, -1e30) / _TEMP)
    sum_exp = jnp.sum(e, axis=1, keepdims=True)
    pos = jnp.sum(e * lab, axis=1, keepdims=True)
    rowv = lax.broadcasted_iota(jnp.int32, (_NPAD, 1), 0) < _N
    r = jnp.where(rowv, pos / sum_exp, 1.0)
    out_ref[...] = -jnp.sum(jnp.log(r), axis=0, keepdims=True) / _N


def _make_unif(interpret=False):
    return pl.pallas_call(
        _unif_body,
        out_shape=jax.ShapeDtypeStruct((1, 1), jnp.float32),
        interpret=interpret,
    )


_NC, _NS = 2, 16
_NW = _NC * _NS
_BPW = _B // _NW
_CH = 128
_NCHUNK = _BPW // _CH


@functools.cache
def _make_sc_gather():
    @functools.partial(
        pl.kernel,
        mesh=plsc.VectorSubcoreMesh(core_axis_name="c", subcore_axis_name="s"),
        out_type=jax.ShapeDtypeStruct((_B, _D), jnp.float32),
        scratch_types=[
            pltpu.VMEM((_CH,), jnp.int32),
            pltpu.VMEM((_CH,), jnp.int32),
            pltpu.VMEM((_CH, _D), jnp.float32),
            pltpu.VMEM((_CH, _D), jnp.float32),
            pltpu.SemaphoreType.DMA,
            pltpu.SemaphoreType.DMA,
        ],
    )
    def _sc_gather(emb_hbm, idx_hbm, out_hbm, i0, i1, r0, r1, s0, s1):
        wid = lax.axis_index("s") * _NC + lax.axis_index("c")
        base = wid * _BPW
        ib, rb, sb = (i0, i1), (r0, r1), (s0, s1)
        handles = {}
        pltpu.sync_copy(idx_hbm.at[pl.ds(base, _CH)], ib[0])
        handles[0] = pltpu.async_copy(emb_hbm.at[ib[0]], rb[0], sb[0])
        for c in range(_NCHUNK):
            r = c % 2
            if c + 1 < _NCHUNK:
                nr = (c + 1) % 2
                pltpu.sync_copy(
                    idx_hbm.at[pl.ds(base + (c + 1) * _CH, _CH)], ib[nr])
                handles[c + 1] = pltpu.async_copy(
                    emb_hbm.at[ib[nr]], rb[nr], sb[nr])
            handles[c].wait()
            pltpu.sync_copy(rb[r], out_hbm.at[pl.ds(base + c * _CH, _CH)])

    return _sc_gather


_dist = _make_dist()
_unif = _make_unif()


def kernel(x, Q, embeddings):
    idx3, dsum = _dist(x, Q.reshape(_G, 1, _TILE), embeddings.T * 2.0)
    idx = idx3.reshape(_B)
    quantized = _make_sc_gather()(embeddings, idx)
    loss = (1.0 + _COMMIT) * dsum[0, 0] / (_B * _D)
    ul = _unif(embeddings, _samp_const())[0, 0]
    return quantized, loss, ul, idx
